# Initial kernel scaffold; baseline (speedup 1.0000x reference)
import jax
import jax.numpy as jnp
from jax.experimental import pallas as pl


def kernel(x, edge_index, node_rankings, W, b):
    raise NotImplementedError("write your pallas kernel here")


# speedup vs baseline: 7.5420x; 7.5420x over previous
import functools

import jax
import jax.numpy as jnp
from jax import lax
from jax.experimental import pallas as pl
from jax.experimental.pallas import tpu as pltpu
from jax.experimental.pallas import tpu_sc as plsc

_K_ACTIVE = 5000
_C = 128
_NCORES = 2
_NSUB = 16
_NW = _NCORES * _NSUB


def _linear_mask_body(x_ref, ranks_ref, wt_ref, b_ref, out_ref):
    h = jnp.dot(x_ref[...], wt_ref[...], preferred_element_type=jnp.float32)
    h = h + b_ref[...]
    active = (ranks_ref[...] <= _K_ACTIVE).astype(jnp.float32)
    out_ref[...] = h * active


def _linear_mask(x, ranks_col, wt, b_row):
    n, d = x.shape
    blk = 2000
    grid = n // blk
    return pl.pallas_call(
        _linear_mask_body,
        grid=(grid,),
        in_specs=[
            pl.BlockSpec((blk, d), lambda i: (i, 0)),
            pl.BlockSpec((blk, 1), lambda i: (i, 0)),
            pl.BlockSpec((d, d), lambda i: (0, 0)),
            pl.BlockSpec((1, d), lambda i: (0, 0)),
        ],
        out_specs=pl.BlockSpec((blk, d), lambda i: (i, 0)),
        out_shape=jax.ShapeDtypeStruct((n, d), jnp.float32),
    )(x, ranks_col, wt, b_row)


def _sum_body(a_ref, b_ref, out_ref):
    out_ref[...] = a_ref[...] + b_ref[...]


def _sum_partials(partials, nacc, d):
    blk = nacc // 4
    return pl.pallas_call(
        _sum_body,
        grid=(4,),
        in_specs=[
            pl.BlockSpec((blk, d), lambda j: (j, 0)),
            pl.BlockSpec((blk, d), lambda j: (j + 4, 0)),
        ],
        out_specs=pl.BlockSpec((blk, d), lambda j: (j, 0)),
        out_shape=jax.ShapeDtypeStruct((nacc, d), jnp.float32),
    )(partials, partials)


def _make_sc_aggregate(n, d, nacc, g):
    rows_per_tile = nacc // _NSUB
    zrows = 64
    mesh = plsc.VectorSubcoreMesh(core_axis_name="c", subcore_axis_name="s")

    @functools.partial(
        pl.kernel,
        out_type=jax.ShapeDtypeStruct((_NCORES * nacc, d), jnp.float32),
        mesh=mesh,
        scratch_types=[
            pltpu.VMEM_SHARED((nacc, d), jnp.float32),
            pltpu.VMEM((_C,), jnp.int32),
            pltpu.VMEM((_C,), jnp.int32),
            pltpu.VMEM((_C, d), jnp.float32),
            pltpu.VMEM((zrows, d), jnp.float32),
            pltpu.SemaphoreType.DMA,
        ],
    )
    def k(h_hbm, src_hbm, dst_hbm, out_hbm, acc, srcbuf, dstbuf, rows, zbuf, sem):
        cid = lax.axis_index("c")
        sid = lax.axis_index("s")
        wid = cid * _NSUB + sid

        def zfill(i, carry):
            zbuf[i // 8, pl.ds((i % 8) * 16, 16)] = jnp.zeros((16,), jnp.float32)
            return carry

        lax.fori_loop(0, zrows * (d // 16), zfill, 0)

        def zcopy(j, carry):
            pltpu.sync_copy(
                zbuf, acc.at[pl.ds(sid * rows_per_tile + j * zrows, zrows)])
            return carry

        lax.fori_loop(0, rows_per_tile // zrows, zcopy, 0)
        plsc.subcore_barrier()

        def step(gi, carry):
            base = (wid * g + gi) * _C
            pltpu.sync_copy(src_hbm.at[pl.ds(base, _C)], srcbuf)
            pltpu.sync_copy(dst_hbm.at[pl.ds(base, _C)], dstbuf)
            pltpu.async_copy(h_hbm.at[srcbuf], rows, sem).wait()
            pltpu.sync_copy(rows, acc.at[dstbuf], add=True)
            return carry

        lax.fori_loop(0, g, step, 0)
        plsc.subcore_barrier()

        r0 = sid * rows_per_tile
        pltpu.sync_copy(
            acc.at[pl.ds(r0, rows_per_tile)],
            out_hbm.at[pl.ds(cid * nacc + r0, rows_per_tile)])

    return k


def kernel(x, edge_index, node_rankings, W, b):
    n, d = x.shape
    e = edge_index.shape[1]

    h_act = _linear_mask(
        x, node_rankings[0][:, None], W.T, b[None, :])

    g = -(-e // (_NW * _C))
    e_pad = _NW * g * _C
    nacc = ((n + _NSUB * 64 - 1) // (_NSUB * 64)) * (_NSUB * 64)
    src = edge_index[0]
    dst = edge_index[1]
    pad = e_pad - e
    src_p = jnp.concatenate([src, jnp.zeros((pad,), jnp.int32)])
    dst_p = jnp.concatenate([dst, jnp.full((pad,), n, jnp.int32)])

    partials = _make_sc_aggregate(n, d, nacc, g)(h_act, src_p, dst_p)
    out = _sum_partials(partials, nacc, d)
    return out[:n]
